# baseline (speedup 1.0000x reference)
import jax
import jax.numpy as jnp
from jax import lax
from jax.experimental import pallas as pl
from jax.experimental.pallas import tpu as pltpu
from jax.experimental.pallas import tpu_sc as plsc

K = 256
ROWS = 64
COLS = 8192
NWORKERS = 32
ROWS_PER_W = ROWS // NWORKERS
NVEC = COLS // 16
MIN32 = -(2 ** 31)
HREG = 4096
CB = COLS + 16


def _srl(x, n):
    if x.ndim == 0:
        return lax.shift_right_logical(x, jnp.int32(n))
    return lax.shift_right_logical(x, jnp.full(x.shape, n, jnp.int32))


def _scalarize(x):
    if x.ndim == 0:
        return x
    return jnp.max(x)


def _sc_body(in_hbm, out_hbm, buf, ukey, cand1, cand2, hist):
    iota = lax.iota(jnp.int32, 16)
    zeros16 = jnp.zeros((16,), jnp.int32)
    ones16 = jnp.ones((16,), jnp.int32)
    min32v = jnp.full((16,), MIN32, jnp.int32)
    R = ROWS_PER_W

    wid = lax.axis_index("s") * 2 + lax.axis_index("c")
    base = wid * R

    pltpu.sync_copy(in_hbm.at[pl.ds(base, R)], buf)

    @plsc.parallel_loop(0, R * 4 * HREG // 16, unroll=8)
    def _(i):
        hist[pl.ds(i * 16, 16)] = zeros16

    def find_digit(kth, hbase):
        def g_body(gq, gt):
            for q in range(4):
                g = gq * 4 + q
                goff = hbase + g * 256
                acc = zeros16
                for j in range(16):
                    acc = acc + hist[pl.ds(goff + j * 16, 16)]
                gt = gt + jnp.where(iota == g, jnp.sum(acc), 0)
            return gt

        gt = lax.fori_loop(0, 4, g_body, zeros16)

        rgt = jnp.flip(gt)
        cg = plsc.cumsum(rgt)
        pc1 = _scalarize(plsc.all_reduce_population_count(cg >= kth))
        i1 = jnp.int32(16) - pc1
        g_star = jnp.int32(15) - i1
        sel1 = iota == i1
        cg_at = jnp.sum(jnp.where(sel1, cg, 0))
        rgt_at = jnp.sum(jnp.where(sel1, rgt, 0))
        above = cg_at - rgt_at

        goff = hbase + g_star * 256
        tot = zeros16
        for j in range(16):
            sj = jnp.sum(hist[pl.ds(goff + j * 16, 16)])
            tot = tot + jnp.where(iota == j, sj, 0)

        racc = jnp.flip(tot)
        c2 = above + plsc.cumsum(racc)
        pc2 = _scalarize(plsc.all_reduce_population_count(c2 >= kth))
        i2 = jnp.int32(16) - pc2
        d_star = g_star * 16 + (jnp.int32(15) - i2)
        sel2 = iota == i2
        cnt_at = jnp.sum(jnp.where(sel2, racc, 0))
        c2_at = jnp.sum(jnp.where(sel2, c2, 0))
        return d_star, kth - (c2_at - cnt_at)

    lane = [[iota + (r * 4 + lvl) * HREG for lvl in range(4)] for r in range(R)]

    @plsc.parallel_loop(0, NVEC, unroll=4)
    def _(i):
        off = i * 16
        for r in range(R):
            v = buf[r, pl.ds(off, 16)]
            u = lax.bitcast_convert_type(v, jnp.int32)
            s = lax.shift_right_arithmetic(u, jnp.full((16,), 31, jnp.int32))
            uk = u ^ (s | min32v)
            ukey[pl.ds(r * COLS + off, 16)] = uk
            dsc = _srl(uk, 20) & 0xFF0
            plsc.addupdate_scatter(hist, [dsc + lane[r][0]], ones16)

    d1, kth = zip(*[find_digit(jnp.int32(K), (r * 4) * HREG)
                    for r in range(R)])
    d1v = [jnp.broadcast_to(d, (16,)) for d in d1]

    @plsc.parallel_loop(0, NVEC, unroll=4, carry=(jnp.int32(0),) * R)
    def n1(i, offc):
        off = i * 16
        out = []
        for r in range(R):
            uk = ukey[pl.ds(r * COLS + off, 16)]
            m = (_srl(uk, 24) & 0xFF) == d1v[r]
            dsc = _srl(uk, 12) & 0xFF0
            plsc.addupdate_scatter(hist, [dsc + lane[r][1]], ones16, mask=m)
            plsc.store_compressed(cand1.at[pl.ds(r * CB + offc[r], 16)],
                                  uk, mask=m)
            out.append(offc[r]
                       + _scalarize(plsc.all_reduce_population_count(m)))
        return tuple(out)

    d2, kth = zip(*[find_digit(kth[r], (r * 4 + 1) * HREG) for r in range(R)])
    d2v = [jnp.broadcast_to(d, (16,)) for d in d2]
    n1v = [jnp.broadcast_to(n, (16,)) for n in n1]

    nv1 = [lax.shift_right_logical(n + 15, jnp.int32(4)) for n in n1]
    nv1max = jnp.maximum(nv1[0], nv1[1]) if R == 2 else nv1[0]

    @plsc.parallel_loop(0, nv1max, unroll=1, carry=(jnp.int32(0),) * R)
    def n2(i, offc):
        off = i * 16
        pos = off + iota
        out = []
        for r in range(R):
            uk = cand1[pl.ds(r * CB + off, 16)]
            m = (pos < n1v[r]) & ((_srl(uk, 16) & 0xFF) == d2v[r])
            dsc = _srl(uk, 4) & 0xFF0
            plsc.addupdate_scatter(hist, [dsc + lane[r][2]], ones16, mask=m)
            plsc.store_compressed(cand2.at[pl.ds(r * CB + offc[r], 16)],
                                  uk, mask=m)
            out.append(offc[r]
                       + _scalarize(plsc.all_reduce_population_count(m)))
        return tuple(out)

    d3, kth = zip(*[find_digit(kth[r], (r * 4 + 2) * HREG) for r in range(R)])
    d3v = [jnp.broadcast_to(d, (16,)) for d in d3]
    n2v = [jnp.broadcast_to(n, (16,)) for n in n2]

    nv2 = [lax.shift_right_logical(n + 15, jnp.int32(4)) for n in n2]
    nv2max = jnp.maximum(nv2[0], nv2[1]) if R == 2 else nv2[0]

    @plsc.parallel_loop(0, nv2max, unroll=1)
    def _(i):
        off = i * 16
        pos = off + iota
        for r in range(R):
            uk = cand2[pl.ds(r * CB + off, 16)]
            m = (pos < n2v[r]) & ((_srl(uk, 8) & 0xFF) == d3v[r])
            dsc = lax.shift_left(uk, jnp.full((16,), 4, jnp.int32)) & 0xFF0
            plsc.addupdate_scatter(hist, [dsc + lane[r][3]], ones16, mask=m)

    d4 = [find_digit(kth[r], (r * 4 + 3) * HREG)[0] for r in range(R)]

    sthr = []
    for r in range(R):
        thr = (lax.shift_left(d1[r], jnp.int32(24))
               | lax.shift_left(d2[r], jnp.int32(16))
               | lax.shift_left(d3[r], jnp.int32(8))
               | d4[r])
        sthr.append(jnp.broadcast_to(thr ^ jnp.int32(MIN32), (16,)))

    @plsc.parallel_loop(0, NVEC, unroll=4)
    def _(i):
        off = i * 16
        for r in range(R):
            uk = ukey[pl.ds(r * COLS + off, 16)]
            keep = (uk ^ min32v) >= sthr[r]
            v = buf[r, pl.ds(off, 16)]
            buf[r, pl.ds(off, 16)] = jnp.where(keep, v, jnp.float32(0))

    pltpu.sync_copy(buf, out_hbm.at[pl.ds(base, R)])


@jax.jit
def kernel(inputs):
    mesh = plsc.VectorSubcoreMesh(
        core_axis_name="c", subcore_axis_name="s",
        num_cores=2, num_subcores=16)
    run = pl.kernel(
        _sc_body,
        out_type=jax.ShapeDtypeStruct((ROWS, COLS), jnp.float32),
        mesh=mesh,
        compiler_params=pltpu.CompilerParams(needs_layout_passes=False),
        scratch_types=[
            pltpu.VMEM((ROWS_PER_W, COLS), jnp.float32),
            pltpu.VMEM((ROWS_PER_W * COLS,), jnp.int32),
            pltpu.VMEM((ROWS_PER_W * CB,), jnp.int32),
            pltpu.VMEM((ROWS_PER_W * CB,), jnp.int32),
            pltpu.VMEM((ROWS_PER_W * 4 * HREG,), jnp.int32),
        ],
    )
    return run(inputs)


# speedup vs baseline: 1.1105x; 1.1105x over previous
import jax
import jax.numpy as jnp
from jax import lax
from jax.experimental import pallas as pl
from jax.experimental.pallas import tpu as pltpu
from jax.experimental.pallas import tpu_sc as plsc

K = 256
ROWS = 64
COLS = 8192
NWORKERS = 32
ROWS_PER_W = ROWS // NWORKERS
NVEC = COLS // 16
MIN32 = -(2 ** 31)
HREG = 4096
CB = COLS + 16


def _srl(x, n):
    if x.ndim == 0:
        return lax.shift_right_logical(x, jnp.int32(n))
    return lax.shift_right_logical(x, jnp.full(x.shape, n, jnp.int32))


def _scalarize(x):
    if x.ndim == 0:
        return x
    return jnp.max(x)


def _sc_body(in_hbm, out_hbm, buf, ukey, cand1, cand2, hist, in_sem, out_sem):
    iota = lax.iota(jnp.int32, 16)
    zeros16 = jnp.zeros((16,), jnp.int32)
    ones16 = jnp.ones((16,), jnp.int32)
    min32v = jnp.full((16,), MIN32, jnp.int32)
    R = ROWS_PER_W

    wid = lax.axis_index("s") * 2 + lax.axis_index("c")
    base = wid * R

    in_copy = pltpu.async_copy(in_hbm.at[pl.ds(base, R)], buf, in_sem)

    @plsc.parallel_loop(0, R * 4 * HREG // 16, unroll=8)
    def _(i):
        hist[pl.ds(i * 16, 16)] = zeros16

    in_copy.wait()

    def find_digit(kth, hbase):
        def g_body(g, gt):
            acc = zeros16
            goff = hbase + g * 256
            for j in range(16):
                acc = acc + hist[pl.ds(goff + j * 16, 16)]
            sg = jnp.sum(acc)
            return gt + jnp.where(iota == g, sg, 0)

        gt = lax.fori_loop(0, 16, g_body, zeros16)

        rgt = jnp.flip(gt)
        cg = plsc.cumsum(rgt)
        pc1 = _scalarize(plsc.all_reduce_population_count(cg >= kth))
        i1 = jnp.int32(16) - pc1
        g_star = jnp.int32(15) - i1
        sel1 = iota == i1
        cg_at = jnp.sum(jnp.where(sel1, cg, 0))
        rgt_at = jnp.sum(jnp.where(sel1, rgt, 0))
        above = cg_at - rgt_at

        goff = hbase + g_star * 256
        tot = zeros16
        for j in range(16):
            sj = jnp.sum(hist[pl.ds(goff + j * 16, 16)])
            tot = tot + jnp.where(iota == j, sj, 0)

        racc = jnp.flip(tot)
        c2 = above + plsc.cumsum(racc)
        pc2 = _scalarize(plsc.all_reduce_population_count(c2 >= kth))
        i2 = jnp.int32(16) - pc2
        d_star = g_star * 16 + (jnp.int32(15) - i2)
        sel2 = iota == i2
        cnt_at = jnp.sum(jnp.where(sel2, racc, 0))
        c2_at = jnp.sum(jnp.where(sel2, c2, 0))
        return d_star, kth - (c2_at - cnt_at)

    lane = [[iota + (r * 4 + lvl) * HREG for lvl in range(4)] for r in range(R)]

    @plsc.parallel_loop(0, NVEC, unroll=4)
    def _(i):
        off = i * 16
        for r in range(R):
            v = buf[r, pl.ds(off, 16)]
            u = lax.bitcast_convert_type(v, jnp.int32)
            s = lax.shift_right_arithmetic(u, jnp.full((16,), 31, jnp.int32))
            uk = u ^ (s | min32v)
            ukey[pl.ds(r * COLS + off, 16)] = uk
            dsc = _srl(uk, 20) & 0xFF0
            plsc.addupdate_scatter(hist, [dsc + lane[r][0]], ones16)

    d1, kth = zip(*[find_digit(jnp.int32(K), (r * 4) * HREG)
                    for r in range(R)])
    d1v = [jnp.broadcast_to(d, (16,)) for d in d1]

    @plsc.parallel_loop(0, NVEC, unroll=4, carry=(jnp.int32(0),) * R)
    def n1(i, offc):
        off = i * 16
        out = []
        for r in range(R):
            uk = ukey[pl.ds(r * COLS + off, 16)]
            m = (_srl(uk, 24) & 0xFF) == d1v[r]
            dsc = _srl(uk, 12) & 0xFF0
            plsc.addupdate_scatter(hist, [dsc + lane[r][1]], ones16, mask=m)
            plsc.store_compressed(cand1.at[pl.ds(r * CB + offc[r], 16)],
                                  uk, mask=m)
            out.append(offc[r]
                       + _scalarize(plsc.all_reduce_population_count(m)))
        return tuple(out)

    d2, kth = zip(*[find_digit(kth[r], (r * 4 + 1) * HREG) for r in range(R)])
    d2v = [jnp.broadcast_to(d, (16,)) for d in d2]
    n1v = [jnp.broadcast_to(n, (16,)) for n in n1]

    nv1 = [lax.shift_right_logical(n + 15, jnp.int32(4)) for n in n1]
    nv1max = jnp.maximum(nv1[0], nv1[1]) if R == 2 else nv1[0]

    @plsc.parallel_loop(0, nv1max, unroll=1, carry=(jnp.int32(0),) * R)
    def n2(i, offc):
        off = i * 16
        pos = off + iota
        out = []
        for r in range(R):
            uk = cand1[pl.ds(r * CB + off, 16)]
            m = (pos < n1v[r]) & ((_srl(uk, 16) & 0xFF) == d2v[r])
            dsc = _srl(uk, 4) & 0xFF0
            plsc.addupdate_scatter(hist, [dsc + lane[r][2]], ones16, mask=m)
            plsc.store_compressed(cand2.at[pl.ds(r * CB + offc[r], 16)],
                                  uk, mask=m)
            out.append(offc[r]
                       + _scalarize(plsc.all_reduce_population_count(m)))
        return tuple(out)

    d3, kth = zip(*[find_digit(kth[r], (r * 4 + 2) * HREG) for r in range(R)])
    d3v = [jnp.broadcast_to(d, (16,)) for d in d3]
    n2v = [jnp.broadcast_to(n, (16,)) for n in n2]

    nv2 = [lax.shift_right_logical(n + 15, jnp.int32(4)) for n in n2]
    nv2max = jnp.maximum(nv2[0], nv2[1]) if R == 2 else nv2[0]

    @plsc.parallel_loop(0, nv2max, unroll=1)
    def _(i):
        off = i * 16
        pos = off + iota
        for r in range(R):
            uk = cand2[pl.ds(r * CB + off, 16)]
            m = (pos < n2v[r]) & ((_srl(uk, 8) & 0xFF) == d3v[r])
            dsc = lax.shift_left(uk, jnp.full((16,), 4, jnp.int32)) & 0xFF0
            plsc.addupdate_scatter(hist, [dsc + lane[r][3]], ones16, mask=m)

    d4 = [find_digit(kth[r], (r * 4 + 3) * HREG)[0] for r in range(R)]

    sthr = []
    for r in range(R):
        thr = (lax.shift_left(d1[r], jnp.int32(24))
               | lax.shift_left(d2[r], jnp.int32(16))
               | lax.shift_left(d3[r], jnp.int32(8))
               | d4[r])
        sthr.append(jnp.broadcast_to(thr ^ jnp.int32(MIN32), (16,)))

    NCHUNK = 2
    CVEC = NVEC // NCHUNK
    out_copies = []
    for c in range(NCHUNK):
        @plsc.parallel_loop(c * CVEC, (c + 1) * CVEC, unroll=4)
        def _(i):
            off = i * 16
            for r in range(R):
                uk = ukey[pl.ds(r * COLS + off, 16)]
                keep = (uk ^ min32v) >= sthr[r]
                v = buf[r, pl.ds(off, 16)]
                buf[r, pl.ds(off, 16)] = jnp.where(keep, v, jnp.float32(0))

        for r in range(R):
            out_copies.append(pltpu.async_copy(
                buf.at[r, pl.ds(c * CVEC * 16, CVEC * 16)],
                out_hbm.at[base + r, pl.ds(c * CVEC * 16, CVEC * 16)],
                out_sem))

    for h in out_copies:
        h.wait()


@jax.jit
def kernel(inputs):
    mesh = plsc.VectorSubcoreMesh(
        core_axis_name="c", subcore_axis_name="s",
        num_cores=2, num_subcores=16)
    run = pl.kernel(
        _sc_body,
        out_type=jax.ShapeDtypeStruct((ROWS, COLS), jnp.float32),
        mesh=mesh,
        compiler_params=pltpu.CompilerParams(needs_layout_passes=False),
        scratch_types=[
            pltpu.VMEM((ROWS_PER_W, COLS), jnp.float32),
            pltpu.VMEM((ROWS_PER_W * COLS,), jnp.int32),
            pltpu.VMEM((ROWS_PER_W * CB,), jnp.int32),
            pltpu.VMEM((ROWS_PER_W * CB,), jnp.int32),
            pltpu.VMEM((ROWS_PER_W * 4 * HREG,), jnp.int32),
            pltpu.SemaphoreType.DMA,
            pltpu.SemaphoreType.DMA,
        ],
    )
    return run(inputs)


# speedup vs baseline: 1.1106x; 1.0001x over previous
import jax
import jax.numpy as jnp
from jax import lax
from jax.experimental import pallas as pl
from jax.experimental.pallas import tpu as pltpu
from jax.experimental.pallas import tpu_sc as plsc

K = 256
ROWS = 64
COLS = 8192
NWORKERS = 32
ROWS_PER_W = ROWS // NWORKERS
NVEC = COLS // 16
MIN32 = -(2 ** 31)
HREG = 4096
CB = COLS + 16


def _srl(x, n):
    if x.ndim == 0:
        return lax.shift_right_logical(x, jnp.int32(n))
    return lax.shift_right_logical(x, jnp.full(x.shape, n, jnp.int32))


def _scalarize(x):
    if x.ndim == 0:
        return x
    return jnp.max(x)


def _sc_body(in_hbm, out_hbm, buf, cand1, cand2, hist, in_sem, out_sem):
    iota = lax.iota(jnp.int32, 16)
    zeros16 = jnp.zeros((16,), jnp.int32)
    ones16 = jnp.ones((16,), jnp.int32)
    min32v = jnp.full((16,), MIN32, jnp.int32)
    R = ROWS_PER_W

    wid = lax.axis_index("s") * 2 + lax.axis_index("c")
    base = wid * R

    in_copy = pltpu.async_copy(in_hbm.at[pl.ds(base, R)], buf, in_sem)

    @plsc.parallel_loop(0, R * 4 * HREG // 16, unroll=8)
    def _(i):
        hist[pl.ds(i * 16, 16)] = zeros16

    in_copy.wait()

    def find_digit(kth, hbase):
        def g_body(g, gt):
            acc = zeros16
            goff = hbase + g * 256
            for j in range(16):
                acc = acc + hist[pl.ds(goff + j * 16, 16)]
            sg = jnp.sum(acc)
            return gt + jnp.where(iota == g, sg, 0)

        gt = lax.fori_loop(0, 16, g_body, zeros16)

        rgt = jnp.flip(gt)
        cg = plsc.cumsum(rgt)
        pc1 = _scalarize(plsc.all_reduce_population_count(cg >= kth))
        i1 = jnp.int32(16) - pc1
        g_star = jnp.int32(15) - i1
        sel1 = iota == i1
        cg_at = jnp.sum(jnp.where(sel1, cg, 0))
        rgt_at = jnp.sum(jnp.where(sel1, rgt, 0))
        above = cg_at - rgt_at

        goff = hbase + g_star * 256
        tot = zeros16
        for j in range(16):
            sj = jnp.sum(hist[pl.ds(goff + j * 16, 16)])
            tot = tot + jnp.where(iota == j, sj, 0)

        racc = jnp.flip(tot)
        c2 = above + plsc.cumsum(racc)
        pc2 = _scalarize(plsc.all_reduce_population_count(c2 >= kth))
        i2 = jnp.int32(16) - pc2
        d_star = g_star * 16 + (jnp.int32(15) - i2)
        sel2 = iota == i2
        cnt_at = jnp.sum(jnp.where(sel2, racc, 0))
        c2_at = jnp.sum(jnp.where(sel2, c2, 0))
        return d_star, kth - (c2_at - cnt_at)

    lane = [[iota + (r * 4 + lvl) * HREG for lvl in range(4)] for r in range(R)]

    @plsc.parallel_loop(0, NVEC, unroll=4)
    def _(i):
        off = i * 16
        for r in range(R):
            v = buf[r, pl.ds(off, 16)]
            u = lax.bitcast_convert_type(v, jnp.int32)
            s = lax.shift_right_arithmetic(u, jnp.full((16,), 31, jnp.int32))
            uk = u ^ (s | min32v)
            dsc = _srl(uk, 20) & 0xFF0
            plsc.addupdate_scatter(hist, [dsc + lane[r][0]], ones16)

    d1, kth = zip(*[find_digit(jnp.int32(K), (r * 4) * HREG)
                    for r in range(R)])
    d1v = [jnp.broadcast_to(d, (16,)) for d in d1]

    @plsc.parallel_loop(0, NVEC, unroll=4, carry=(jnp.int32(0),) * R)
    def n1(i, offc):
        off = i * 16
        out = []
        for r in range(R):
            v = buf[r, pl.ds(off, 16)]
            u = lax.bitcast_convert_type(v, jnp.int32)
            s_ = lax.shift_right_arithmetic(u, jnp.full((16,), 31, jnp.int32))
            uk = u ^ (s_ | min32v)
            m = (_srl(uk, 24) & 0xFF) == d1v[r]
            dsc = _srl(uk, 12) & 0xFF0
            plsc.addupdate_scatter(hist, [dsc + lane[r][1]], ones16, mask=m)
            plsc.store_compressed(cand1.at[pl.ds(r * CB + offc[r], 16)],
                                  uk, mask=m)
            out.append(offc[r]
                       + _scalarize(plsc.all_reduce_population_count(m)))
        return tuple(out)

    d2, kth = zip(*[find_digit(kth[r], (r * 4 + 1) * HREG) for r in range(R)])
    d2v = [jnp.broadcast_to(d, (16,)) for d in d2]
    n1v = [jnp.broadcast_to(n, (16,)) for n in n1]

    nv1 = [lax.shift_right_logical(n + 15, jnp.int32(4)) for n in n1]
    nv1max = jnp.maximum(nv1[0], nv1[1]) if R == 2 else nv1[0]

    @plsc.parallel_loop(0, nv1max, unroll=1, carry=(jnp.int32(0),) * R)
    def n2(i, offc):
        off = i * 16
        pos = off + iota
        out = []
        for r in range(R):
            uk = cand1[pl.ds(r * CB + off, 16)]
            m = (pos < n1v[r]) & ((_srl(uk, 16) & 0xFF) == d2v[r])
            dsc = _srl(uk, 4) & 0xFF0
            plsc.addupdate_scatter(hist, [dsc + lane[r][2]], ones16, mask=m)
            plsc.store_compressed(cand2.at[pl.ds(r * CB + offc[r], 16)],
                                  uk, mask=m)
            out.append(offc[r]
                       + _scalarize(plsc.all_reduce_population_count(m)))
        return tuple(out)

    d3, kth = zip(*[find_digit(kth[r], (r * 4 + 2) * HREG) for r in range(R)])
    d3v = [jnp.broadcast_to(d, (16,)) for d in d3]
    n2v = [jnp.broadcast_to(n, (16,)) for n in n2]

    nv2 = [lax.shift_right_logical(n + 15, jnp.int32(4)) for n in n2]
    nv2max = jnp.maximum(nv2[0], nv2[1]) if R == 2 else nv2[0]

    @plsc.parallel_loop(0, nv2max, unroll=1)
    def _(i):
        off = i * 16
        pos = off + iota
        for r in range(R):
            uk = cand2[pl.ds(r * CB + off, 16)]
            m = (pos < n2v[r]) & ((_srl(uk, 8) & 0xFF) == d3v[r])
            dsc = lax.shift_left(uk, jnp.full((16,), 4, jnp.int32)) & 0xFF0
            plsc.addupdate_scatter(hist, [dsc + lane[r][3]], ones16, mask=m)

    d4 = [find_digit(kth[r], (r * 4 + 3) * HREG)[0] for r in range(R)]

    sthr = []
    for r in range(R):
        thr = (lax.shift_left(d1[r], jnp.int32(24))
               | lax.shift_left(d2[r], jnp.int32(16))
               | lax.shift_left(d3[r], jnp.int32(8))
               | d4[r])
        sthr.append(jnp.broadcast_to(thr ^ jnp.int32(MIN32), (16,)))

    NCHUNK = 2
    CVEC = NVEC // NCHUNK
    out_copies = []
    for c in range(NCHUNK):
        @plsc.parallel_loop(c * CVEC, (c + 1) * CVEC, unroll=4)
        def _(i):
            off = i * 16
            for r in range(R):
                v = buf[r, pl.ds(off, 16)]
                u = lax.bitcast_convert_type(v, jnp.int32)
                s_ = lax.shift_right_arithmetic(
                    u, jnp.full((16,), 31, jnp.int32))
                keep = (u ^ (s_ & jnp.int32(0x7FFFFFFF))) >= sthr[r]
                buf[r, pl.ds(off, 16)] = jnp.where(keep, v, jnp.float32(0))

        for r in range(R):
            out_copies.append(pltpu.async_copy(
                buf.at[r, pl.ds(c * CVEC * 16, CVEC * 16)],
                out_hbm.at[base + r, pl.ds(c * CVEC * 16, CVEC * 16)],
                out_sem))

    for h in out_copies:
        h.wait()


@jax.jit
def kernel(inputs):
    mesh = plsc.VectorSubcoreMesh(
        core_axis_name="c", subcore_axis_name="s",
        num_cores=2, num_subcores=16)
    run = pl.kernel(
        _sc_body,
        out_type=jax.ShapeDtypeStruct((ROWS, COLS), jnp.float32),
        mesh=mesh,
        compiler_params=pltpu.CompilerParams(needs_layout_passes=False),
        scratch_types=[
            pltpu.VMEM((ROWS_PER_W, COLS), jnp.float32),
            pltpu.VMEM((ROWS_PER_W * CB,), jnp.int32),
            pltpu.VMEM((ROWS_PER_W * CB,), jnp.int32),
            pltpu.VMEM((ROWS_PER_W * 4 * HREG,), jnp.int32),
            pltpu.SemaphoreType.DMA,
            pltpu.SemaphoreType.DMA,
        ],
    )
    return run(inputs)
